# baseline (speedup 1.0000x reference)
import functools

import jax
import jax.numpy as jnp
from jax.experimental import pallas as pl
from jax.experimental.pallas import tpu as pltpu

N_ROWS = 50000
DIM = 1408
BLOCK = 1000
NB = N_ROWS // BLOCK
TOPK = 10
ALPHA = 0.8
NEG_INF = float("-inf")
INT_MAX = 2**31 - 1


_ACOS_COEFFS = (
    1.5707963050, -0.2145988016, 0.0889789874, -0.0501743046,
    0.0308918810, -0.0170881256, 0.0066700901, -0.0012624911,
)


def _acos(x):
    xa = jnp.abs(x)
    p = jnp.float32(_ACOS_COEFFS[-1])
    for c in _ACOS_COEFFS[-2::-1]:
        p = p * xa + jnp.float32(c)
    t = jnp.sqrt(jnp.maximum(1.0 - xa, 0.0)) * p
    return jnp.where(x >= 0.0, t, jnp.float32(3.14159265358979) - t)


def _reranker_body(a_ref, b_ref, v_ref, vals_ref, idx_ref, q_s, sims_s):
    i = pl.program_id(0)

    @pl.when(i == 0)
    def _compute_query():
        a = a_ref[...]
        b = b_ref[...]
        na = jnp.sqrt(jnp.sum(a * a))
        nb = jnp.sqrt(jnp.sum(b * b))
        ua = a / na
        ub = b / nb
        dot = jnp.clip(jnp.sum(ua * ub), -1.0, 1.0)
        theta = _acos(dot)
        st = jnp.sin(theta)
        st_safe = jnp.where(st == 0.0, 1.0, st)
        w_a = jnp.sin((1.0 - ALPHA) * theta) / st_safe
        w_b = jnp.sin(ALPHA * theta) / st_safe
        q = jnp.where(st == 0.0, a, w_a * a + w_b * b)
        qn = q / jnp.sqrt(jnp.sum(q * q))
        q_s[...] = jnp.broadcast_to(qn, (8, DIM))

    vb = v_ref[...]
    sq = jnp.sum(vb * vb, axis=1, keepdims=True)
    rinv = 1.0 / jnp.sqrt(sq)
    vnb = (vb * rinv).astype(jnp.bfloat16)
    qb = q_s[...].astype(jnp.bfloat16)
    dot = jax.lax.dot_general(
        qb, vnb, (((1,), (1,)), ((), ())),
        preferred_element_type=jnp.float32,
    )
    sims_s[pl.ds(i, 1), :] = dot[0:1, :]

    @pl.when(i == NB - 1)
    def _topk():
        s = sims_s[...]
        row = jax.lax.broadcasted_iota(jnp.int32, (NB, BLOCK), 0)
        col = jax.lax.broadcasted_iota(jnp.int32, (NB, BLOCK), 1)
        gidx = row * BLOCK + col
        lane = jax.lax.broadcasted_iota(jnp.int32, (1, 128), 1)
        vals_acc = jnp.full((1, 128), NEG_INF, dtype=jnp.float32)
        idx_acc = jnp.zeros((1, 128), dtype=jnp.int32)
        for k in range(TOPK):
            m = jnp.max(s)
            cand = jnp.where(s == m, gidx, INT_MAX)
            amin = jnp.min(cand)
            vals_acc = jnp.where(lane == k, m, vals_acc)
            idx_acc = jnp.where(lane == k, amin, idx_acc)
            s = jnp.where(gidx == amin, NEG_INF, s)
        vals_ref[...] = vals_acc
        idx_ref[...] = idx_acc


@jax.jit
def kernel(emb_a, emb_b, video_embeddings):
    a2 = emb_a.reshape(1, DIM)
    b2 = emb_b.reshape(1, DIM)
    vals, idx = pl.pallas_call(
        _reranker_body,
        grid=(NB,),
        in_specs=[
            pl.BlockSpec((1, DIM), lambda i: (0, 0)),
            pl.BlockSpec((1, DIM), lambda i: (0, 0)),
            pl.BlockSpec((BLOCK, DIM), lambda i: (i, 0)),
        ],
        out_specs=[
            pl.BlockSpec((1, 128), lambda i: (0, 0)),
            pl.BlockSpec((1, 128), lambda i: (0, 0)),
        ],
        out_shape=[
            jax.ShapeDtypeStruct((1, 128), jnp.float32),
            jax.ShapeDtypeStruct((1, 128), jnp.int32),
        ],
        scratch_shapes=[
            pltpu.VMEM((8, DIM), jnp.float32),
            pltpu.VMEM((NB, BLOCK), jnp.float32),
        ],
    )(a2, b2, video_embeddings)
    return vals[0, :TOPK], idx[0, :TOPK]


# speedup vs baseline: 1.1618x; 1.1618x over previous
import functools

import jax
import jax.numpy as jnp
from jax.experimental import pallas as pl
from jax.experimental.pallas import tpu as pltpu

N_ROWS = 50000
DIM = 1408
BLOCK = 2000
NB = N_ROWS // BLOCK
TOPK = 10
ALPHA = 0.8
NEG_INF = float("-inf")
INT_MAX = 2**31 - 1


_ACOS_COEFFS = (
    1.5707963050, -0.2145988016, 0.0889789874, -0.0501743046,
    0.0308918810, -0.0170881256, 0.0066700901, -0.0012624911,
)


def _acos(x):
    xa = jnp.abs(x)
    p = jnp.float32(_ACOS_COEFFS[-1])
    for c in _ACOS_COEFFS[-2::-1]:
        p = p * xa + jnp.float32(c)
    t = jnp.sqrt(jnp.maximum(1.0 - xa, 0.0)) * p
    return jnp.where(x >= 0.0, t, jnp.float32(3.14159265358979) - t)


def _reranker_body(a_ref, b_ref, v_ref, vals_ref, idx_ref, q_s, sims_s):
    i = pl.program_id(0)

    @pl.when(i == 0)
    def _compute_query():
        a = a_ref[...]
        b = b_ref[...]
        na = jnp.sqrt(jnp.sum(a * a))
        nb = jnp.sqrt(jnp.sum(b * b))
        ua = a / na
        ub = b / nb
        dot = jnp.clip(jnp.sum(ua * ub), -1.0, 1.0)
        theta = _acos(dot)
        st = jnp.sin(theta)
        st_safe = jnp.where(st == 0.0, 1.0, st)
        w_a = jnp.sin((1.0 - ALPHA) * theta) / st_safe
        w_b = jnp.sin(ALPHA * theta) / st_safe
        q = jnp.where(st == 0.0, a, w_a * a + w_b * b)
        qn = q / jnp.sqrt(jnp.sum(q * q))
        q_s[...] = jnp.broadcast_to(qn, (8, DIM))

    vb = v_ref[...]
    sq = jnp.sum(vb * vb, axis=1, keepdims=True)
    rinv = 1.0 / jnp.sqrt(sq)
    vnb = (vb * rinv).astype(jnp.bfloat16)
    qb = q_s[...].astype(jnp.bfloat16)
    dot = jax.lax.dot_general(
        qb, vnb, (((1,), (1,)), ((), ())),
        preferred_element_type=jnp.float32,
    )
    sims_s[pl.ds(i, 1), :] = dot[0:1, :]

    @pl.when(i == NB - 1)
    def _topk():
        s = sims_s[...]
        row = jax.lax.broadcasted_iota(jnp.int32, (NB, BLOCK), 0)
        col = jax.lax.broadcasted_iota(jnp.int32, (NB, BLOCK), 1)
        gidx = row * BLOCK + col
        lane = jax.lax.broadcasted_iota(jnp.int32, (1, 128), 1)
        vals_acc = jnp.full((1, 128), NEG_INF, dtype=jnp.float32)
        idx_acc = jnp.zeros((1, 128), dtype=jnp.int32)
        for k in range(TOPK):
            m = jnp.max(s)
            cand = jnp.where(s == m, gidx, INT_MAX)
            amin = jnp.min(cand)
            vals_acc = jnp.where(lane == k, m, vals_acc)
            idx_acc = jnp.where(lane == k, amin, idx_acc)
            s = jnp.where(gidx == amin, NEG_INF, s)
        vals_ref[...] = vals_acc
        idx_ref[...] = idx_acc


@jax.jit
def kernel(emb_a, emb_b, video_embeddings):
    a2 = emb_a.reshape(1, DIM)
    b2 = emb_b.reshape(1, DIM)
    vals, idx = pl.pallas_call(
        _reranker_body,
        grid=(NB,),
        in_specs=[
            pl.BlockSpec((1, DIM), lambda i: (0, 0)),
            pl.BlockSpec((1, DIM), lambda i: (0, 0)),
            pl.BlockSpec((BLOCK, DIM), lambda i: (i, 0)),
        ],
        out_specs=[
            pl.BlockSpec((1, 128), lambda i: (0, 0)),
            pl.BlockSpec((1, 128), lambda i: (0, 0)),
        ],
        out_shape=[
            jax.ShapeDtypeStruct((1, 128), jnp.float32),
            jax.ShapeDtypeStruct((1, 128), jnp.int32),
        ],
        scratch_shapes=[
            pltpu.VMEM((8, DIM), jnp.float32),
            pltpu.VMEM((NB, BLOCK), jnp.float32),
        ],
    )(a2, b2, video_embeddings)
    return vals[0, :TOPK], idx[0, :TOPK]
